# baseline (speedup 1.0000x reference)
import jax
import jax.numpy as jnp
from jax.experimental import pallas as pl
from jax.experimental.pallas import tpu as pltpu

_EPSILON = 0.02
_GAMMA = 0.5
_N = 4194304

_BN = 65536
_CH = 512
_NCH = _BN // _CH
_STEPS = _N // _BN

_DN = (((1,), (0,)), ((), ()))


def _loss_body(d_ref, c_ref, t_ref, o_ref, acc_ce, acc_sq):
    j = pl.program_id(0)

    @pl.when(j == 0)
    def _init():
        acc_ce[...] = jnp.zeros_like(acc_ce)
        acc_sq[...] = jnp.zeros_like(acc_sq)

    ar = jax.lax.broadcasted_iota(jnp.int32, (8, 8), 0)
    ac = jax.lax.broadcasted_iota(jnp.int32, (8, 8), 1)
    amat = (2.0 * jnp.where((ac >= ar) & (ac <= 3), 1.0, 0.0)
            + jnp.where(ac == ar - 1, 1.0, 0.0))
    kr = jax.lax.broadcasted_iota(jnp.int32, (8, _CH), 0)
    kvec = jnp.where(kr == 0, 0.0, jnp.where(kr <= 4, 1.0, -1.0))
    wc = jax.lax.broadcasted_iota(jnp.int32, (1, 8), 1)
    w5 = jnp.where(wc <= 4, 1.0, 0.0)
    zero3 = jnp.zeros((3, _CH), dtype=jnp.float32)
    zero4 = jnp.zeros((4, _CH), dtype=jnp.float32)

    ce_tot = acc_ce[...]
    sq_tot = acc_sq[...]
    for k in range(_NCH):
        sl = pl.ds(k * _CH, _CH)
        d = d_ref[:, sl]
        t = t_ref[:, sl]
        c = c_ref[:, sl]

        mf = jnp.where(t >= _EPSILON, 1.0, 0.0)
        m8 = jnp.concatenate([mf, zero4], axis=0)
        cmat = jax.lax.dot_general(amat, m8, _DN,
                                   preferred_element_type=jnp.float32)
        onehot = cmat == kvec

        d8 = jnp.concatenate([d, zero3], axis=0)
        e8 = jnp.exp(d8)
        s8 = jnp.where(onehot, d8, 0.0)

        both = jnp.concatenate([e8, s8], axis=1)
        red = jax.lax.dot_general(w5, both, _DN,
                                  precision=jax.lax.Precision.HIGHEST,
                                  preferred_element_type=jnp.float32)
        se = red[:, :_CH]
        d_sel = red[:, _CH:]

        ce_tot = ce_tot + (jnp.log(se) - d_sel)
        diff = c - t
        sq_tot = sq_tot + diff * diff

    acc_ce[...] = ce_tot
    acc_sq[...] = sq_tot

    @pl.when(j == _STEPS - 1)
    def _fin():
        ce = jnp.sum(acc_ce[...])
        sq = jnp.sum(acc_sq[...])
        loss = ce * ((1.0 - _GAMMA) / _N) + sq * (_GAMMA / (4.0 * _N))
        o_ref[...] = loss.reshape(1, 1, 1)


def kernel(decision, cost, target_rcosts):
    parts = pl.pallas_call(
        _loss_body,
        grid=(_STEPS,),
        in_specs=[
            pl.BlockSpec((5, _BN), lambda j: (0, j)),
            pl.BlockSpec((4, _BN), lambda j: (0, j)),
            pl.BlockSpec((4, _BN), lambda j: (0, j)),
        ],
        out_specs=pl.BlockSpec((1, 1, 1), lambda j: (0, 0, 0)),
        out_shape=jax.ShapeDtypeStruct((1, 1, 1), jnp.float32),
        scratch_shapes=[
            pltpu.VMEM((1, _CH), jnp.float32),
            pltpu.VMEM((4, _CH), jnp.float32),
        ],
        compiler_params=pltpu.CompilerParams(
            dimension_semantics=("arbitrary",),
        ),
        name="routing_loss",
    )(decision.T, cost.T, target_rcosts.T)
    return parts.reshape(())


# speedup vs baseline: 8.1993x; 8.1993x over previous
import jax
import jax.numpy as jnp
from jax.experimental import pallas as pl
from jax.experimental.pallas import tpu as pltpu

_EPSILON = 0.02
_GAMMA = 0.5
_N = 4194304

_BN = 65536
_CH = 512
_NCH = _BN // _CH
_STEPS = _N // _BN


def _loss_body(d_ref, c_ref, t_ref, o_ref, acc_ce, acc_sq):
    j = pl.program_id(0)

    @pl.when(j == 0)
    def _init():
        acc_ce[...] = jnp.zeros_like(acc_ce)
        acc_sq[...] = jnp.zeros_like(acc_sq)

    zero3 = jnp.zeros((3, _CH), dtype=jnp.float32)
    zero4 = jnp.zeros((4, _CH), dtype=jnp.float32)
    row4 = jax.lax.broadcasted_iota(jnp.int32, (4, _CH), 0).astype(jnp.float32) + 1.0
    row5 = jax.lax.broadcasted_iota(jnp.int32, (5, _CH), 0).astype(jnp.float32)

    ce_tot = acc_ce[...]
    sq_tot = acc_sq[...]
    for k in range(_NCH):
        sl = pl.ds(k * _CH, _CH)
        d = d_ref[:, sl]
        t = t_ref[:, sl]
        c = c_ref[:, sl]

        e8 = jnp.concatenate([jnp.exp(d), zero3], axis=0)
        se = jnp.sum(e8, axis=0, keepdims=True)

        m8 = jnp.concatenate([jnp.where(t >= _EPSILON, row4, 0.0), zero4], axis=0)
        jusm = jnp.max(m8, axis=0, keepdims=True)

        s8 = jnp.concatenate([jnp.where(row5 == jusm, d, 0.0), zero3], axis=0)
        d_sel = jnp.sum(s8, axis=0, keepdims=True)

        ce_tot = ce_tot + (jnp.log(se) - d_sel)
        diff = c - t
        sq_tot = sq_tot + diff * diff

    acc_ce[...] = ce_tot
    acc_sq[...] = sq_tot

    @pl.when(j == _STEPS - 1)
    def _fin():
        ce = jnp.sum(acc_ce[...])
        sq = jnp.sum(acc_sq[...])
        loss = ce * ((1.0 - _GAMMA) / _N) + sq * (_GAMMA / (4.0 * _N))
        o_ref[...] = loss.reshape(1, 1, 1)


def kernel(decision, cost, target_rcosts):
    parts = pl.pallas_call(
        _loss_body,
        grid=(_STEPS,),
        in_specs=[
            pl.BlockSpec((5, _BN), lambda j: (0, j)),
            pl.BlockSpec((4, _BN), lambda j: (0, j)),
            pl.BlockSpec((4, _BN), lambda j: (0, j)),
        ],
        out_specs=pl.BlockSpec((1, 1, 1), lambda j: (0, 0, 0)),
        out_shape=jax.ShapeDtypeStruct((1, 1, 1), jnp.float32),
        scratch_shapes=[
            pltpu.VMEM((1, _CH), jnp.float32),
            pltpu.VMEM((4, _CH), jnp.float32),
        ],
        compiler_params=pltpu.CompilerParams(
            dimension_semantics=("arbitrary",),
        ),
        name="routing_loss",
    )(decision.T, cost.T, target_rcosts.T)
    return parts.reshape(())


# speedup vs baseline: 8.2009x; 1.0002x over previous
import jax
import jax.numpy as jnp
from jax.experimental import pallas as pl
from jax.experimental.pallas import tpu as pltpu

_EPSILON = 0.02
_GAMMA = 0.5
_N = 4194304

_BN = 131072
_CH = 1024
_NCH = _BN // _CH
_STEPS = _N // _BN


def _loss_body(d_ref, c_ref, t_ref, o_ref, acc_ce, acc_sq):
    j = pl.program_id(0)

    @pl.when(j == 0)
    def _init():
        acc_ce[...] = jnp.zeros_like(acc_ce)
        acc_sq[...] = jnp.zeros_like(acc_sq)

    zero3 = jnp.zeros((3, _CH), dtype=jnp.float32)
    zero4 = jnp.zeros((4, _CH), dtype=jnp.float32)
    row4 = jax.lax.broadcasted_iota(jnp.int32, (4, _CH), 0).astype(jnp.float32) + 1.0
    row5 = jax.lax.broadcasted_iota(jnp.int32, (5, _CH), 0).astype(jnp.float32)

    ce_tot = acc_ce[...]
    sq_tot = acc_sq[...]
    for k in range(_NCH):
        sl = pl.ds(k * _CH, _CH)
        d = d_ref[:, sl]
        t = t_ref[:, sl]
        c = c_ref[:, sl]

        e8 = jnp.concatenate([jnp.exp(d), zero3], axis=0)
        se = jnp.sum(e8, axis=0, keepdims=True)

        m8 = jnp.concatenate([jnp.where(t >= _EPSILON, row4, 0.0), zero4], axis=0)
        jusm = jnp.max(m8, axis=0, keepdims=True)

        s8 = jnp.concatenate([jnp.where(row5 == jusm, d, 0.0), zero3], axis=0)
        d_sel = jnp.sum(s8, axis=0, keepdims=True)

        ce_tot = ce_tot + (jnp.log(se) - d_sel)
        diff = c - t
        sq_tot = sq_tot + diff * diff

    acc_ce[...] = ce_tot
    acc_sq[...] = sq_tot

    @pl.when(j == _STEPS - 1)
    def _fin():
        ce = jnp.sum(acc_ce[...])
        sq = jnp.sum(acc_sq[...])
        loss = ce * ((1.0 - _GAMMA) / _N) + sq * (_GAMMA / (4.0 * _N))
        o_ref[...] = loss.reshape(1, 1, 1)


def kernel(decision, cost, target_rcosts):
    parts = pl.pallas_call(
        _loss_body,
        grid=(_STEPS,),
        in_specs=[
            pl.BlockSpec((5, _BN), lambda j: (0, j)),
            pl.BlockSpec((4, _BN), lambda j: (0, j)),
            pl.BlockSpec((4, _BN), lambda j: (0, j)),
        ],
        out_specs=pl.BlockSpec((1, 1, 1), lambda j: (0, 0, 0)),
        out_shape=jax.ShapeDtypeStruct((1, 1, 1), jnp.float32),
        scratch_shapes=[
            pltpu.VMEM((1, _CH), jnp.float32),
            pltpu.VMEM((4, _CH), jnp.float32),
        ],
        compiler_params=pltpu.CompilerParams(
            dimension_semantics=("arbitrary",),
        ),
        name="routing_loss",
    )(decision.T, cost.T, target_rcosts.T)
    return parts.reshape(())
